# baseline (speedup 1.0000x reference)
import functools

import jax
import jax.numpy as jnp
from jax import lax
from jax.experimental import pallas as pl
from jax.experimental.pallas import tpu as pltpu
from jax.experimental.pallas import tpu_sc as plsc

_VOCAB = 1000000
_EMBED_DIM = 64
_BATCH = 16384

_NUM_WORKERS = 32
_ROWS_PER_WORKER = _BATCH // _NUM_WORKERS
_CHUNK = 128
_CHUNKS = _ROWS_PER_WORKER // _CHUNK
_GROUPS = _CHUNK // 16


def _embed_lookup(indices, table2):
    mesh = plsc.VectorSubcoreMesh(core_axis_name="c", subcore_axis_name="s")

    @functools.partial(
        pl.kernel,
        out_type=jax.ShapeDtypeStruct((_BATCH, _EMBED_DIM), jnp.float32),
        mesh=mesh,
        scratch_types=[
            pltpu.VMEM((_ROWS_PER_WORKER,), jnp.int32),
            pltpu.VMEM((_ROWS_PER_WORKER,), jnp.int32),
            pltpu.VMEM((_CHUNK, 128), jnp.float32),
            pltpu.VMEM((_CHUNK, _EMBED_DIM), jnp.float32),
            pltpu.SemaphoreType.DMA,
        ],
        compiler_params=pltpu.CompilerParams(needs_layout_passes=False),
    )
    def body(idx_hbm, table_hbm, out_hbm, idx_v, tidx_v, rows_v, outc_v, sem):
        wid = lax.axis_index("s") * 2 + lax.axis_index("c")
        base = wid * _ROWS_PER_WORKER
        pltpu.sync_copy(idx_hbm.at[pl.ds(base, _ROWS_PER_WORKER)], idx_v)

        def make_pair_idx(m, carry):
            v = idx_v[pl.ds(m * 16, 16)]
            tidx_v[pl.ds(m * 16, 16)] = jnp.right_shift(v, 1)
            return carry

        lax.fori_loop(0, _ROWS_PER_WORKER // 16, make_pair_idx, 0)

        def chunk_step(cc, carry):
            pltpu.async_copy(
                table_hbm.at[tidx_v.at[pl.ds(cc * _CHUNK, _CHUNK)]],
                rows_v,
                sem,
            ).wait()
            for g in range(_GROUPS):
                i16 = idx_v[pl.ds(cc * _CHUNK + g * 16, 16)]
                h16 = jnp.bitwise_and(i16, 1) * _EMBED_DIM
                k16 = lax.iota(jnp.int32, 16) + g * 16
                for col in range(_EMBED_DIM):
                    cvec = jnp.full((16,), col, jnp.int32)
                    v = plsc.load_gather(rows_v, [k16, h16 + cvec])
                    plsc.store_scatter(outc_v, [k16, cvec], v)
            pltpu.sync_copy(outc_v, out_hbm.at[pl.ds(base + cc * _CHUNK, _CHUNK)])
            return carry

        lax.fori_loop(0, _CHUNKS, chunk_step, 0)

    return body(indices, table2)


def kernel(indices, embed):
    table2 = embed.reshape(_VOCAB // 2, 2 * _EMBED_DIM)
    return _embed_lookup(indices.astype(jnp.int32), table2)


# speedup vs baseline: 1.2328x; 1.2328x over previous
import functools

import jax
import jax.numpy as jnp
from jax import lax
from jax.experimental import pallas as pl
from jax.experimental.pallas import tpu as pltpu
from jax.experimental.pallas import tpu_sc as plsc

_VOCAB = 1000000
_EMBED_DIM = 64
_BATCH = 16384

_NUM_WORKERS = 32
_SEG = _BATCH // _NUM_WORKERS
_NBLK = (_VOCAB + 127) // 128
_FULLBLK = _VOCAB // 128
_TAIL0 = _FULLBLK * 128
_OFFPAD = 7936


def _embed_lookup_sorted(sidx, off, table_cm, tail_tab):
    mesh = plsc.VectorSubcoreMesh(core_axis_name="c", subcore_axis_name="s")

    @functools.partial(
        pl.kernel,
        out_type=jax.ShapeDtypeStruct((_BATCH, _EMBED_DIM), jnp.float32),
        mesh=mesh,
        scratch_types=[
            pltpu.VMEM((_BATCH,), jnp.int32),
            pltpu.VMEM((_OFFPAD,), jnp.int32),
            pltpu.VMEM((_EMBED_DIM, _EMBED_DIM), jnp.float32),
            pltpu.VMEM((_EMBED_DIM, 128), jnp.float32),
            pltpu.VMEM((_SEG, _EMBED_DIM), jnp.float32),
        ],
        compiler_params=pltpu.CompilerParams(needs_layout_passes=False),
    )
    def body(sidx_hbm, off_hbm, tab_hbm, tail_hbm, out_hbm, sv, offv, tailv, blk, outb):
        wid = lax.axis_index("s") * 2 + lax.axis_index("c")
        base = wid * _SEG
        pltpu.sync_copy(sidx_hbm, sv)
        pltpu.sync_copy(off_hbm, offv)
        pltpu.sync_copy(tail_hbm, tailv)

        v_first = sv[pl.ds(base, 16)][0]
        v_last = sv[pl.ds(base + _SEG - 16, 16)][15]
        js = lax.shift_right_logical(v_first, 7)
        je = lax.shift_right_logical(v_last, 7)
        e16 = lax.iota(jnp.int32, 16)
        z16 = jnp.zeros((16,), jnp.int32)

        def blk_step(j, carry):
            lo = jnp.maximum(offv[pl.ds(j, 16)][0], base)
            hi = jnp.minimum(offv[pl.ds(j + 1, 16)][0], base + _SEG)

            @pl.when(hi > lo)
            def _():
                joff = pl.multiple_of(
                    jnp.minimum(j, _FULLBLK - 1) * 128, 128
                )
                pltpu.sync_copy(tab_hbm.at[:, pl.ds(joff, 128)], blk)

                def pos_step(p, c2):
                    v = sv[pl.ds(p, 16)][0]
                    cmain = z16 + jnp.bitwise_and(v, 127)
                    ctail = z16 + jnp.clip(v - _TAIL0, 0, _EMBED_DIM - 1)
                    is_tail = jnp.broadcast_to(v >= _TAIL0, (16,))
                    for q in range(4):
                        xm = plsc.load_gather(blk, [e16 + q * 16, cmain])
                        xt = plsc.load_gather(tailv, [e16 + q * 16, ctail])
                        outb[p - base, pl.ds(q * 16, 16)] = jnp.where(is_tail, xt, xm)
                    return c2

                lax.fori_loop(lo, hi, pos_step, 0)

            return carry

        lax.fori_loop(js, je + 1, blk_step, 0)
        pltpu.sync_copy(outb, out_hbm.at[pl.ds(base, _SEG)])

    return body(sidx, off, table_cm, tail_tab)


def kernel(indices, embed):
    idx32 = indices.astype(jnp.int32)
    order = jnp.argsort(idx32)
    sidx = jnp.take(idx32, order)
    inv = jnp.zeros((_BATCH,), jnp.int32).at[order].set(
        jnp.arange(_BATCH, dtype=jnp.int32)
    )
    starts = jnp.arange(_OFFPAD, dtype=jnp.int32) * 128
    off = jnp.searchsorted(sidx, starts).astype(jnp.int32)
    tail_tab = embed[_TAIL0:].T
    out_sorted = _embed_lookup_sorted(sidx, off, embed.T, tail_tab)
    return jnp.take(out_sorted, inv, axis=0)


# speedup vs baseline: 2.7296x; 2.2142x over previous
import functools

import jax
import jax.numpy as jnp
from jax import lax
from jax.experimental import pallas as pl
from jax.experimental.pallas import tpu as pltpu
from jax.experimental.pallas import tpu_sc as plsc

_VOCAB = 1000000
_EMBED_DIM = 64
_BATCH = 16384

_NUM_WORKERS = 32
_SEG = _BATCH // _NUM_WORKERS
_FULLBLK = _VOCAB // 128
_TAIL0 = _FULLBLK * 128
_SPAN = 256
_MAXJ = _FULLBLK - 2
_RCAP = _SEG + 16


def _embed_lookup_sorted(sidx, table_cm, tail_tab):
    mesh = plsc.VectorSubcoreMesh(core_axis_name="c", subcore_axis_name="s")

    @functools.partial(
        pl.kernel,
        out_type=jax.ShapeDtypeStruct((_BATCH, _EMBED_DIM), jnp.float32),
        mesh=mesh,
        scratch_types=[
            pltpu.VMEM((_SEG + 16,), jnp.int32),
            pltpu.VMEM((_RCAP,), jnp.int32),
            pltpu.VMEM((_RCAP,), jnp.int32),
            pltpu.VMEM((_EMBED_DIM, _EMBED_DIM), jnp.float32),
            pltpu.VMEM((_EMBED_DIM, _SPAN), jnp.float32),
            pltpu.VMEM((_EMBED_DIM, _SPAN), jnp.float32),
            pltpu.VMEM((_SEG, _EMBED_DIM), jnp.float32),
            pltpu.SemaphoreType.DMA,
            pltpu.SemaphoreType.DMA,
        ],
        compiler_params=pltpu.CompilerParams(needs_layout_passes=False),
    )
    def body(sidx_hbm, tab_hbm, tail_hbm, out_hbm,
             sv, rblk, rpos, tailv, bufa, bufb, outb, sema, semb):
        wid = lax.axis_index("s") * 2 + lax.axis_index("c")
        base = wid * _SEG
        pltpu.sync_copy(sidx_hbm.at[pl.ds(base, _SEG)], sv.at[pl.ds(0, _SEG)])
        pltpu.sync_copy(tail_hbm, tailv)

        e16 = lax.iota(jnp.int32, 16)
        z16 = jnp.zeros((16,), jnp.int32)
        lane0 = e16 == 0

        def emit(slot, blk, pos):
            plsc.store_scatter(rblk, [z16 + slot], z16 + blk, mask=lane0)
            plsc.store_scatter(rpos, [z16 + slot], z16 + pos, mask=lane0)

        v0 = sv[pl.ds(0, 16)][0]
        j0 = jnp.minimum(lax.shift_right_logical(v0, 7), _MAXJ)
        emit(0, j0, 0)

        def walk(p, carry):
            n, cur = carry
            v = sv[pl.ds(p, 16)][0]
            j = jnp.minimum(lax.shift_right_logical(v, 7), _MAXJ)
            new = j > cur + 1

            @pl.when(new)
            def _():
                emit(n, j, p)

            return jnp.where(new, n + 1, n), jnp.where(new, j, cur)

        n_runs, _unused = lax.fori_loop(1, _SEG, walk, (jnp.int32(1), j0))
        emit(n_runs, 0, _SEG)

        def span_off(i):
            jb = rblk[pl.ds(i, 16)][0]
            return jb, pl.multiple_of(jb * 128, 128)

        def fetch(i, buf, sem):
            _jb, off = span_off(i)
            pltpu.async_copy(tab_hbm.at[:, pl.ds(off, _SPAN)], buf, sem)

        def drain(buf, sem):
            pltpu.make_async_copy(tab_hbm.at[:, pl.ds(0, _SPAN)], buf, sem).wait()

        def extract(i, buf):
            jb, _off = span_off(i)
            p_lo = rpos[pl.ds(i, 16)][0]
            p_hi = rpos[pl.ds(i + 1, 16)][0]
            cbase = jb * 128

            def pos_step(p, c2):
                v = sv[pl.ds(p, 16)][0]
                cmain = z16 + jnp.clip(v - cbase, 0, _SPAN - 1)
                ctail = z16 + jnp.clip(v - _TAIL0, 0, _EMBED_DIM - 1)
                is_tail = jnp.broadcast_to(v >= _TAIL0, (16,))
                for q in range(4):
                    xm = plsc.load_gather(buf, [e16 + q * 16, cmain])
                    xt = plsc.load_gather(tailv, [e16 + q * 16, ctail])
                    outb[p, pl.ds(q * 16, 16)] = jnp.where(is_tail, xt, xm)
                return c2

            lax.fori_loop(p_lo, p_hi, pos_step, 0)

        fetch(0, bufa, sema)

        def pair_step(m, carry):
            i0 = m * 2
            i1 = i0 + 1

            @pl.when(i0 < n_runs)
            def _():
                drain(bufa, sema)

                @pl.when(i1 < n_runs)
                def _():
                    fetch(i1, bufb, semb)

                extract(i0, bufa)

            @pl.when(i1 < n_runs)
            def _():
                drain(bufb, semb)

                @pl.when(i1 + 1 < n_runs)
                def _():
                    fetch(i1 + 1, bufa, sema)

                extract(i1, bufb)

            return carry

        lax.fori_loop(0, (n_runs + 1) // 2, pair_step, 0)
        pltpu.sync_copy(outb, out_hbm.at[pl.ds(base, _SEG)])

    return body(sidx, table_cm, tail_tab)


def kernel(indices, embed):
    idx32 = indices.astype(jnp.int32)
    order = jnp.argsort(idx32)
    sidx = jnp.take(idx32, order)
    inv = jnp.zeros((_BATCH,), jnp.int32).at[order].set(
        jnp.arange(_BATCH, dtype=jnp.int32)
    )
    tail_tab = embed[_TAIL0:].T
    out_sorted = _embed_lookup_sorted(sidx, embed.T, tail_tab)
    return jnp.take(out_sorted, inv, axis=0)


# speedup vs baseline: 4.0472x; 1.4827x over previous
import functools

import jax
import jax.numpy as jnp
from jax import lax
from jax.experimental import pallas as pl
from jax.experimental.pallas import tpu as pltpu
from jax.experimental.pallas import tpu_sc as plsc

_VOCAB = 1000000
_EMBED_DIM = 64
_BATCH = 16384

_NUM_WORKERS = 32
_SEG = _BATCH // _NUM_WORKERS
_FULLBLK = _VOCAB // 128
_TAIL0 = _FULLBLK * 128
_SPAN = 256
_MAXJ = _FULLBLK - 2
_RCAP = _SEG + 16


def _embed_lookup_sorted(sidx, table_cm, tail_tab):
    mesh = plsc.VectorSubcoreMesh(core_axis_name="c", subcore_axis_name="s")

    @functools.partial(
        pl.kernel,
        out_type=jax.ShapeDtypeStruct((_BATCH, _EMBED_DIM), jnp.float32),
        mesh=mesh,
        scratch_types=[
            pltpu.VMEM((_SEG + 16,), jnp.int32),
            pltpu.VMEM((_RCAP,), jnp.int32),
            pltpu.VMEM((_RCAP,), jnp.int32),
            pltpu.VMEM((_EMBED_DIM, _EMBED_DIM), jnp.float32),
            pltpu.VMEM((_EMBED_DIM, _SPAN), jnp.float32),
            pltpu.VMEM((_EMBED_DIM, _SPAN), jnp.float32),
            pltpu.VMEM((_EMBED_DIM, _SPAN), jnp.float32),
            pltpu.VMEM((_SEG, _EMBED_DIM), jnp.float32),
            pltpu.SemaphoreType.DMA,
            pltpu.SemaphoreType.DMA,
            pltpu.SemaphoreType.DMA,
        ],
        compiler_params=pltpu.CompilerParams(needs_layout_passes=False),
    )
    def body(sidx_hbm, tab_hbm, tail_hbm, out_hbm,
             sv, rblk, rpos, tailv, bufa, bufb, bufc, outb, sema, semb, semc):
        wid = lax.axis_index("s") * 2 + lax.axis_index("c")
        base = wid * _SEG
        pltpu.sync_copy(sidx_hbm.at[pl.ds(base, _SEG)], sv.at[pl.ds(0, _SEG)])
        pltpu.sync_copy(tail_hbm, tailv)

        e16 = lax.iota(jnp.int32, 16)
        z16 = jnp.zeros((16,), jnp.int32)
        lane0 = e16 == 0

        def emit(slot, blk, pos):
            plsc.store_scatter(rblk, [z16 + slot], z16 + blk, mask=lane0)
            plsc.store_scatter(rpos, [z16 + slot], z16 + pos, mask=lane0)

        v0 = sv[pl.ds(0, 16)][0]
        j0 = jnp.minimum(lax.shift_right_logical(v0, 7), _MAXJ)
        emit(0, j0, 0)

        def walk(p, carry):
            n, cur = carry
            v = sv[pl.ds(p, 16)][0]
            j = jnp.minimum(lax.shift_right_logical(v, 7), _MAXJ)
            new = j > cur + 1

            @pl.when(new)
            def _():
                emit(n, j, p)

            return jnp.where(new, n + 1, n), jnp.where(new, j, cur)

        n_runs, _unused = lax.fori_loop(1, _SEG, walk, (jnp.int32(1), j0))
        emit(n_runs, 0, _SEG)

        def span_off(i):
            jb = rblk[pl.ds(i, 16)][0]
            return jb, pl.multiple_of(jb * 128, 128)

        def fetch(i, buf, sem):
            _jb, off = span_off(i)
            pltpu.async_copy(tab_hbm.at[:, pl.ds(off, _SPAN)], buf, sem)

        def drain(buf, sem):
            pltpu.make_async_copy(tab_hbm.at[:, pl.ds(0, _SPAN)], buf, sem).wait()

        def extract(i, buf):
            jb, _off = span_off(i)
            p_lo = rpos[pl.ds(i, 16)][0]
            p_hi = rpos[pl.ds(i + 1, 16)][0]
            cbase = jb * 128

            def pos_step(p, c2):
                v = sv[pl.ds(p, 16)][0]
                cmain = z16 + jnp.clip(v - cbase, 0, _SPAN - 1)
                ctail = z16 + jnp.clip(v - _TAIL0, 0, _EMBED_DIM - 1)
                is_tail = jnp.broadcast_to(v >= _TAIL0, (16,))
                for q in range(4):
                    xm = plsc.load_gather(buf, [e16 + q * 16, cmain])
                    xt = plsc.load_gather(tailv, [e16 + q * 16, ctail])
                    outb[p, pl.ds(q * 16, 16)] = jnp.where(is_tail, xt, xm)
                return c2

            lax.fori_loop(p_lo, p_hi, pos_step, 0)

        ring = [(bufa, sema), (bufb, semb), (bufc, semc)]
        fetch(0, bufa, sema)

        @pl.when(n_runs > 1)
        def _():
            fetch(1, bufb, semb)

        def tri_step(m, carry):
            for k in range(3):
                i = m * 3 + k
                buf, sem = ring[k]
                nbuf, nsem = ring[(k + 2) % 3]

                @pl.when(i < n_runs)
                def _(i=i, buf=buf, sem=sem, nbuf=nbuf, nsem=nsem):
                    drain(buf, sem)

                    @pl.when(i + 2 < n_runs)
                    def _():
                        fetch(i + 2, nbuf, nsem)

                    extract(i, buf)

            return carry

        lax.fori_loop(0, (n_runs + 2) // 3, tri_step, 0)
        pltpu.sync_copy(outb, out_hbm.at[pl.ds(base, _SEG)])

    return body(sidx, table_cm, tail_tab)


def kernel(indices, embed):
    idx32 = indices.astype(jnp.int32)
    order = jnp.argsort(idx32)
    sidx = jnp.take(idx32, order)
    inv = jnp.argsort(order).astype(jnp.int32)
    tail_tab = embed[_TAIL0:].T
    out_sorted = _embed_lookup_sorted(sidx, embed.T, tail_tab)
    return out_sorted.at[inv].get(mode="promise_in_bounds", unique_indices=True)


# speedup vs baseline: 4.1920x; 1.0358x over previous
import functools

import jax
import jax.numpy as jnp
from jax import lax
from jax.experimental import pallas as pl
from jax.experimental.pallas import tpu as pltpu
from jax.experimental.pallas import tpu_sc as plsc

_VOCAB = 1000000
_EMBED_DIM = 64
_BATCH = 16384

_NUM_WORKERS = 32
_SEG = _BATCH // _NUM_WORKERS
_FULLBLK = _VOCAB // 128
_TAIL0 = _FULLBLK * 128
_SPAN = 256
_MAXJ = _FULLBLK - 2
_RCAP = _SEG + 16


def _embed_lookup_sorted(sidx, table_cm, tail_tab):
    mesh = plsc.VectorSubcoreMesh(core_axis_name="c", subcore_axis_name="s")

    @functools.partial(
        pl.kernel,
        out_type=jax.ShapeDtypeStruct((_BATCH, _EMBED_DIM), jnp.float32),
        mesh=mesh,
        scratch_types=[
            pltpu.VMEM((_SEG + 16,), jnp.int32),
            pltpu.VMEM((_RCAP,), jnp.int32),
            pltpu.VMEM((_RCAP,), jnp.int32),
            pltpu.VMEM((_EMBED_DIM, _EMBED_DIM), jnp.float32),
            pltpu.VMEM((_EMBED_DIM, _SPAN), jnp.float32),
            pltpu.VMEM((_EMBED_DIM, _SPAN), jnp.float32),
            pltpu.VMEM((_EMBED_DIM, _SPAN), jnp.float32),
            pltpu.VMEM((_SEG, _EMBED_DIM), jnp.float32),
            pltpu.SemaphoreType.DMA,
            pltpu.SemaphoreType.DMA,
            pltpu.SemaphoreType.DMA,
        ],
        compiler_params=pltpu.CompilerParams(needs_layout_passes=False),
    )
    def body(sidx_hbm, tab_hbm, tail_hbm, out_hbm,
             sv, rblk, rpos, tailv, bufa, bufb, bufc, outb, sema, semb, semc):
        wid = lax.axis_index("s") * 2 + lax.axis_index("c")
        base = wid * _SEG
        pltpu.sync_copy(sidx_hbm.at[pl.ds(base, _SEG)], sv.at[pl.ds(0, _SEG)])
        pltpu.sync_copy(tail_hbm, tailv)

        e16 = lax.iota(jnp.int32, 16)
        z16 = jnp.zeros((16,), jnp.int32)
        lane0 = e16 == 0

        def emit(slot, blk, pos):
            plsc.store_scatter(rblk, [z16 + slot], z16 + blk, mask=lane0)
            plsc.store_scatter(rpos, [z16 + slot], z16 + pos, mask=lane0)

        v0 = sv[pl.ds(0, 16)][0]
        j0 = jnp.minimum(lax.shift_right_logical(v0, 7), _MAXJ)
        emit(0, j0, 0)

        def walk(p, carry):
            n, cur = carry
            v = sv[pl.ds(p, 16)][0]
            j = jnp.minimum(lax.shift_right_logical(v, 7), _MAXJ)
            new = j > cur + 1

            @pl.when(new)
            def _():
                emit(n, j, p)

            return jnp.where(new, n + 1, n), jnp.where(new, j, cur)

        n_runs, _unused = lax.fori_loop(1, _SEG, walk, (jnp.int32(1), j0))
        emit(n_runs, 0, _SEG)

        def span_off(i):
            jb = rblk[pl.ds(i, 16)][0]
            return jb, pl.multiple_of(jb * 128, 128)

        def fetch(i, buf, sem):
            _jb, off = span_off(i)
            pltpu.async_copy(tab_hbm.at[:, pl.ds(off, _SPAN)], buf, sem)

        def drain(buf, sem):
            pltpu.make_async_copy(tab_hbm.at[:, pl.ds(0, _SPAN)], buf, sem).wait()

        def extract(i, buf):
            jb, _off = span_off(i)
            p_lo = rpos[pl.ds(i, 16)][0]
            p_hi = rpos[pl.ds(i + 1, 16)][0]
            cbase = jb * 128

            def pos_step(p, c2):
                v = sv[pl.ds(p, 16)][0]
                cmain = z16 + jnp.clip(v - cbase, 0, _SPAN - 1)
                ctail = z16 + jnp.clip(v - _TAIL0, 0, _EMBED_DIM - 1)
                is_tail = jnp.broadcast_to(v >= _TAIL0, (16,))
                for q in range(4):
                    xm = plsc.load_gather(buf, [e16 + q * 16, cmain])
                    xt = plsc.load_gather(tailv, [e16 + q * 16, ctail])
                    outb[p, pl.ds(q * 16, 16)] = jnp.where(is_tail, xt, xm)
                return c2

            lax.fori_loop(p_lo, p_hi, pos_step, 0)

        ring = [(bufa, sema), (bufb, semb), (bufc, semc)]
        fetch(0, bufa, sema)

        @pl.when(n_runs > 1)
        def _():
            fetch(1, bufb, semb)

        def tri_step(m, carry):
            for k in range(3):
                i = m * 3 + k
                buf, sem = ring[k]
                nbuf, nsem = ring[(k + 2) % 3]

                @pl.when(i < n_runs)
                def _(i=i, buf=buf, sem=sem, nbuf=nbuf, nsem=nsem):
                    drain(buf, sem)

                    @pl.when(i + 2 < n_runs)
                    def _():
                        fetch(i + 2, nbuf, nsem)

                    extract(i, buf)

            return carry

        lax.fori_loop(0, (n_runs + 2) // 3, tri_step, 0)
        pltpu.sync_copy(outb, out_hbm.at[pl.ds(base, _SEG)])

    return body(sidx, table_cm, tail_tab)


def kernel(indices, embed):
    idx32 = indices.astype(jnp.int32)
    sidx, order = lax.sort(
        (idx32, jnp.arange(_BATCH, dtype=jnp.int32)), num_keys=1
    )
    inv = jnp.argsort(order).astype(jnp.int32)
    tail_tab = embed[_TAIL0:].T
    out_sorted = _embed_lookup_sorted(sidx, embed.T, tail_tab)
    return out_sorted.at[inv].get(mode="promise_in_bounds", unique_indices=True)


# speedup vs baseline: 4.2204x; 1.0068x over previous
import functools

import jax
import jax.numpy as jnp
from jax import lax
from jax.experimental import pallas as pl
from jax.experimental.pallas import tpu as pltpu
from jax.experimental.pallas import tpu_sc as plsc

_VOCAB = 1000000
_EMBED_DIM = 64
_BATCH = 16384

_NUM_WORKERS = 32
_SEG = _BATCH // _NUM_WORKERS
_FULLBLK = _VOCAB // 128
_TAIL0 = _FULLBLK * 128
_SPAN = 256
_MAXJ = _FULLBLK - 2
_RCAP = _SEG + 16


def _embed_lookup_sorted(sidx, table_cm, tail_tab):
    mesh = plsc.VectorSubcoreMesh(core_axis_name="c", subcore_axis_name="s")

    @functools.partial(
        pl.kernel,
        out_type=jax.ShapeDtypeStruct((_BATCH, _EMBED_DIM), jnp.float32),
        mesh=mesh,
        scratch_types=[
            pltpu.VMEM((_SEG + 16,), jnp.int32),
            pltpu.VMEM((_RCAP,), jnp.int32),
            pltpu.VMEM((_RCAP,), jnp.int32),
            pltpu.VMEM((_EMBED_DIM, _EMBED_DIM), jnp.float32),
            pltpu.VMEM((_EMBED_DIM, _SPAN), jnp.float32),
            pltpu.VMEM((_EMBED_DIM, _SPAN), jnp.float32),
            pltpu.VMEM((_EMBED_DIM, _SPAN), jnp.float32),
            pltpu.VMEM((_SEG, _EMBED_DIM), jnp.float32),
            pltpu.SemaphoreType.DMA,
            pltpu.SemaphoreType.DMA,
            pltpu.SemaphoreType.DMA,
        ],
        compiler_params=pltpu.CompilerParams(needs_layout_passes=False),
    )
    def body(sidx_hbm, tab_hbm, tail_hbm, out_hbm,
             sv, rblk, rpos, tailv, bufa, bufb, bufc, outb, sema, semb, semc):
        wid = lax.axis_index("s") * 2 + lax.axis_index("c")
        base = wid * _SEG
        pltpu.sync_copy(sidx_hbm.at[pl.ds(base, _SEG)], sv.at[pl.ds(0, _SEG)])
        pltpu.sync_copy(tail_hbm, tailv)

        e16 = lax.iota(jnp.int32, 16)
        z16 = jnp.zeros((16,), jnp.int32)
        lane0 = e16 == 0

        def emit(slot, blk, pos):
            plsc.store_scatter(rblk, [z16 + slot], z16 + blk, mask=lane0)
            plsc.store_scatter(rpos, [z16 + slot], z16 + pos, mask=lane0)

        v0 = sv[pl.ds(0, 16)][0]
        j0 = jnp.minimum(lax.shift_right_logical(v0, 7), _MAXJ)
        emit(0, j0, 0)
        pltpu.async_copy(
            tab_hbm.at[:, pl.ds(pl.multiple_of(j0 * 128, 128), _SPAN)], bufa, sema
        )

        def walk(p, carry):
            n, cur = carry
            v = sv[pl.ds(p, 16)][0]
            j = jnp.minimum(lax.shift_right_logical(v, 7), _MAXJ)
            new = j > cur + 1

            @pl.when(new)
            def _():
                emit(n, j, p)

            return jnp.where(new, n + 1, n), jnp.where(new, j, cur)

        n_runs, _unused = lax.fori_loop(1, _SEG, walk, (jnp.int32(1), j0))
        emit(n_runs, 0, _SEG)

        def span_off(i):
            jb = rblk[pl.ds(i, 16)][0]
            return jb, pl.multiple_of(jb * 128, 128)

        def fetch(i, buf, sem):
            _jb, off = span_off(i)
            pltpu.async_copy(tab_hbm.at[:, pl.ds(off, _SPAN)], buf, sem)

        def drain(buf, sem):
            pltpu.make_async_copy(tab_hbm.at[:, pl.ds(0, _SPAN)], buf, sem).wait()

        def extract(i, buf):
            jb, _off = span_off(i)
            p_lo = rpos[pl.ds(i, 16)][0]
            p_hi = rpos[pl.ds(i + 1, 16)][0]
            cbase = jb * 128

            def pos_step(p, c2):
                v = sv[pl.ds(p, 16)][0]
                cmain = z16 + jnp.clip(v - cbase, 0, _SPAN - 1)
                ctail = z16 + jnp.clip(v - _TAIL0, 0, _EMBED_DIM - 1)
                is_tail = jnp.broadcast_to(v >= _TAIL0, (16,))
                for q in range(4):
                    xm = plsc.load_gather(buf, [e16 + q * 16, cmain])
                    xt = plsc.load_gather(tailv, [e16 + q * 16, ctail])
                    outb[p, pl.ds(q * 16, 16)] = jnp.where(is_tail, xt, xm)
                return c2

            lax.fori_loop(p_lo, p_hi, pos_step, 0)

        ring = [(bufa, sema), (bufb, semb), (bufc, semc)]

        @pl.when(n_runs > 1)
        def _():
            fetch(1, bufb, semb)

        def tri_step(m, carry):
            for k in range(3):
                i = m * 3 + k
                buf, sem = ring[k]
                nbuf, nsem = ring[(k + 2) % 3]

                @pl.when(i < n_runs)
                def _(i=i, buf=buf, sem=sem, nbuf=nbuf, nsem=nsem):
                    drain(buf, sem)

                    @pl.when(i + 2 < n_runs)
                    def _():
                        fetch(i + 2, nbuf, nsem)

                    extract(i, buf)

            return carry

        lax.fori_loop(0, (n_runs + 2) // 3, tri_step, 0)
        pltpu.sync_copy(outb, out_hbm.at[pl.ds(base, _SEG)])

    return body(sidx, table_cm, tail_tab)


def kernel(indices, embed):
    idx32 = indices.astype(jnp.int32)
    sidx, order = lax.sort(
        (idx32, jnp.arange(_BATCH, dtype=jnp.int32)), num_keys=1
    )
    inv = jnp.argsort(order).astype(jnp.int32)
    tail_tab = embed[_TAIL0:].T
    out_sorted = _embed_lookup_sorted(sidx, embed.T, tail_tab)
    return out_sorted.at[inv].get(mode="promise_in_bounds", unique_indices=True)
